# baseline (speedup 1.0000x reference)
import jax
import jax.numpy as jnp
from jax.experimental import pallas as pl


_BS = 128


def _add_body(in_ref, pos_ref, out_ref):
    out_ref[...] = in_ref[...] + pos_ref[...][None, :, :]


def kernel(inputs, pos_table):
    B, S, D = inputs.shape
    return pl.pallas_call(
        _add_body,
        grid=(S // _BS,),
        in_specs=[
            pl.BlockSpec((B, _BS, D), lambda i: (0, i, 0)),
            pl.BlockSpec((_BS, D), lambda i: (i, 0)),
        ],
        out_specs=pl.BlockSpec((B, _BS, D), lambda i: (0, i, 0)),
        out_shape=jax.ShapeDtypeStruct((B, S, D), inputs.dtype),
    )(inputs, pos_table)


# speedup vs baseline: 1.0666x; 1.0666x over previous
import jax
import jax.numpy as jnp
from jax.experimental import pallas as pl


_BS = 512


def _add_body(in_ref, pos_ref, out_ref):
    out_ref[...] = in_ref[...] + pos_ref[...][None, :, :]


def kernel(inputs, pos_table):
    B, S, D = inputs.shape
    return pl.pallas_call(
        _add_body,
        grid=(S // _BS,),
        in_specs=[
            pl.BlockSpec((B, _BS, D), lambda i: (0, i, 0)),
            pl.BlockSpec((_BS, D), lambda i: (i, 0)),
        ],
        out_specs=pl.BlockSpec((B, _BS, D), lambda i: (0, i, 0)),
        out_shape=jax.ShapeDtypeStruct((B, S, D), inputs.dtype),
    )(inputs, pos_table)
